# baseline (speedup 1.0000x reference)
import jax
import jax.numpy as jnp
from jax.experimental import pallas as pl
from jax.experimental.pallas import tpu as pltpu
from jax.experimental.pallas import tpu_sc as plsc

_SEP = 102
_N = 32768
_D = 2048
_BLK_A = 1024
_BLK_B = 256


_SUBCORES = 32
_PER_SUB = _N // _SUBCORES
_LANES = 16


def _sc_scan(x):
    xr = x.reshape(_SUBCORES, _PER_SUB)

    @pl.kernel(
        out_type=jax.ShapeDtypeStruct((_SUBCORES, _LANES), jnp.int32),
        mesh=plsc.VectorSubcoreMesh(core_axis_name="c", subcore_axis_name="s"),
        scratch_types=[
            pltpu.VMEM((_PER_SUB,), jnp.int32),
            pltpu.VMEM((_LANES,), jnp.int32),
        ],
    )
    def scan_kernel(x_hbm, o_hbm, xv, accv):
        k = jax.lax.axis_index("c") * 16 + jax.lax.axis_index("s")
        pltpu.sync_copy(x_hbm.at[k], xv)
        lane = jax.lax.iota(jnp.int32, _LANES)
        base = k * _PER_SUB

        def body(i, acc):
            v = xv[pl.ds(i * _LANES, _LANES)]
            pos = base + i * _LANES + lane
            return jnp.minimum(acc, jnp.where(v == _SEP, pos, _N))

        accv[...] = jax.lax.fori_loop(
            0, _PER_SUB // _LANES, body, jnp.full((_LANES,), _N, jnp.int32)
        )
        pltpu.sync_copy(accv, o_hbm.at[k])

    return scan_kernel(xr)


def _pass_a_body(tab1_ref, out_ref):
    i = pl.program_id(0)

    @pl.when(i < 4)
    def _():
        out_ref[...] = jnp.broadcast_to(tab1_ref[...], out_ref.shape)


def _pass_a(table):
    tab1 = table[1:2, :]
    return pl.pallas_call(
        _pass_a_body,
        grid=(_N // _BLK_A,),
        in_specs=[pl.BlockSpec(tab1.shape, lambda i: (0, 0))],
        out_specs=pl.BlockSpec((_BLK_A, _D), lambda i: (i, 0)),
        out_shape=jax.ShapeDtypeStruct((_N, _D), table.dtype),
    )(tab1)


def _pass_b_body(t_ref, tab_ref, buf_ref, out_ref):
    i = pl.program_id(0)
    t = t_ref[0]
    last = jnp.maximum(pl.cdiv(t, _BLK_B) - 1, 0)

    @pl.when(i <= last)
    def _():
        row = i * _BLK_B + jax.lax.broadcasted_iota(jnp.int32, (_BLK_B, _D), 0)
        t0 = jnp.broadcast_to(tab_ref[0:1, :], (_BLK_B, _D))
        t1 = jnp.broadcast_to(tab_ref[1:2, :], (_BLK_B, _D))
        out_ref[...] = jnp.where(row < t, t0, t1)


def _pass_b(t, table, buf):
    grid_spec = pltpu.PrefetchScalarGridSpec(
        num_scalar_prefetch=1,
        grid=(_N // _BLK_B,),
        in_specs=[
            pl.BlockSpec(table.shape, lambda i, t: (0, 0)),
            pl.BlockSpec(memory_space=pl.ANY),
        ],
        out_specs=pl.BlockSpec(
            (_BLK_B, _D),
            lambda i, t: (jnp.minimum(i, jnp.maximum(pl.cdiv(t[0], _BLK_B) - 1, 0)), 0),
        ),
    )
    return pl.pallas_call(
        _pass_b_body,
        grid_spec=grid_spec,
        out_shape=jax.ShapeDtypeStruct((_N, _D), table.dtype),
        input_output_aliases={2: 0},
    )(t, table, buf)


def kernel(x, table):
    partials = _sc_scan(x)
    t = jnp.min(partials).reshape(1)
    buf = _pass_a(table)
    out = _pass_b(t, table, buf)
    return out.reshape(x.shape + (table.shape[1],))


# speedup vs baseline: 1.0173x; 1.0173x over previous
import dataclasses

import jax
import jax.numpy as jnp
from jax.experimental import pallas as pl
from jax.experimental.pallas import tpu as pltpu
from jax.experimental.pallas import tpu_sc as plsc

_SEP = 102
_N = 32768
_D = 2048
_BLK_A = 1024
_BLK_B = 2048


_LANES = 16


def _sc_scan(x):
    xr = x.reshape(1, _N)

    cp = pltpu.CompilerParams()
    if "needs_layout_passes" in pltpu.CompilerParams.__dataclass_fields__:
        cp = dataclasses.replace(cp, needs_layout_passes=False)

    @pl.kernel(
        out_type=jax.ShapeDtypeStruct((_LANES,), jnp.int32),
        mesh=plsc.VectorSubcoreMesh(core_axis_name="c", subcore_axis_name="s"),
        compiler_params=cp,
        scratch_types=[
            pltpu.VMEM((_N,), jnp.int32),
            pltpu.VMEM((_LANES,), jnp.int32),
        ],
    )
    def scan_kernel(x_hbm, o_hbm, xv, accv):
        c = jax.lax.axis_index("c")
        s = jax.lax.axis_index("s")

        @pl.when((c == 0) & (s == 0))
        def _():
            pltpu.sync_copy(x_hbm.at[0], xv)
            lane = jax.lax.iota(jnp.int32, _LANES)

            def body(i, acc):
                v = xv[pl.ds(i * _LANES, _LANES)]
                pos = i * _LANES + lane
                return jnp.minimum(acc, jnp.where(v == _SEP, pos, _N))

            acc = jax.lax.fori_loop(
                0, _N // _LANES, body, jnp.full((_LANES,), _N, jnp.int32)
            )
            accv[...] = jnp.full((_LANES,), jnp.min(acc), jnp.int32)
            pltpu.sync_copy(accv, o_hbm)

    return scan_kernel(xr)


def _pass_a_body(tab1_ref, out_ref):
    i = pl.program_id(0)

    @pl.when(i < 4)
    def _():
        out_ref[...] = jnp.broadcast_to(tab1_ref[...], out_ref.shape)


def _pass_a(table):
    tab1 = table[1:2, :]
    return pl.pallas_call(
        _pass_a_body,
        grid=(_N // _BLK_A,),
        in_specs=[pl.BlockSpec(tab1.shape, lambda i: (0, 0))],
        out_specs=pl.BlockSpec((_BLK_A, _D), lambda i: (i, 0)),
        out_shape=jax.ShapeDtypeStruct((_N, _D), table.dtype),
    )(tab1)


def _pass_b_body(t_ref, tab_ref, buf_ref, out_ref):
    i = pl.program_id(0)
    t = t_ref[0]
    last = jnp.maximum(pl.cdiv(t, _BLK_B) - 1, 0)

    @pl.when(i <= last)
    def _():
        row = i * _BLK_B + jax.lax.broadcasted_iota(jnp.int32, (_BLK_B, _D), 0)
        t0 = jnp.broadcast_to(tab_ref[0:1, :], (_BLK_B, _D))
        t1 = jnp.broadcast_to(tab_ref[1:2, :], (_BLK_B, _D))
        out_ref[...] = jnp.where(row < t, t0, t1)


def _pass_b(t, table, buf):
    grid_spec = pltpu.PrefetchScalarGridSpec(
        num_scalar_prefetch=1,
        grid=(_N // _BLK_B,),
        in_specs=[
            pl.BlockSpec(table.shape, lambda i, t: (0, 0)),
            pl.BlockSpec(memory_space=pl.ANY),
        ],
        out_specs=pl.BlockSpec(
            (_BLK_B, _D),
            lambda i, t: (jnp.minimum(i, jnp.maximum(pl.cdiv(t[0], _BLK_B) - 1, 0)), 0),
        ),
    )
    return pl.pallas_call(
        _pass_b_body,
        grid_spec=grid_spec,
        out_shape=jax.ShapeDtypeStruct((_N, _D), table.dtype),
        input_output_aliases={2: 0},
    )(t, table, buf)


def kernel(x, table):
    t = _sc_scan(x)
    buf = _pass_a(table)
    out = _pass_b(t, table, buf)
    return out.reshape(x.shape + (table.shape[1],))


# speedup vs baseline: 1.0383x; 1.0207x over previous
import dataclasses

import jax
import jax.numpy as jnp
from jax.experimental import pallas as pl
from jax.experimental.pallas import tpu as pltpu
from jax.experimental.pallas import tpu_sc as plsc

_SEP = 102
_N = 32768
_D = 2048
_BLK_A = 1024
_BLK_B = 512


_LANES = 16


def _sc_scan(x):
    xr = x.reshape(1, _N)

    cp = pltpu.CompilerParams()
    if "needs_layout_passes" in pltpu.CompilerParams.__dataclass_fields__:
        cp = dataclasses.replace(cp, needs_layout_passes=False)

    @pl.kernel(
        out_type=jax.ShapeDtypeStruct((_LANES,), jnp.int32),
        mesh=plsc.VectorSubcoreMesh(core_axis_name="c", subcore_axis_name="s"),
        compiler_params=cp,
        scratch_types=[
            pltpu.VMEM((_N,), jnp.int32),
            pltpu.VMEM((_LANES,), jnp.int32),
        ],
    )
    def scan_kernel(x_hbm, o_hbm, xv, accv):
        c = jax.lax.axis_index("c")
        s = jax.lax.axis_index("s")

        @pl.when((c == 0) & (s == 0))
        def _():
            pltpu.sync_copy(x_hbm.at[0], xv)
            lane = jax.lax.iota(jnp.int32, _LANES)

            def body(i, acc):
                v = xv[pl.ds(i * _LANES, _LANES)]
                pos = i * _LANES + lane
                return jnp.minimum(acc, jnp.where(v == _SEP, pos, _N))

            acc = jax.lax.fori_loop(
                0, _N // _LANES, body, jnp.full((_LANES,), _N, jnp.int32)
            )
            accv[...] = jnp.full((_LANES,), jnp.min(acc), jnp.int32)
            pltpu.sync_copy(accv, o_hbm)

    return scan_kernel(xr)


def _pass_a_body(tab1_ref, out_ref):
    i = pl.program_id(0)

    @pl.when(i < 4)
    def _():
        out_ref[...] = jnp.broadcast_to(tab1_ref[...], out_ref.shape)


def _pass_a(table):
    tab1 = table[1:2, :]
    return pl.pallas_call(
        _pass_a_body,
        grid=(_N // _BLK_A,),
        in_specs=[pl.BlockSpec(tab1.shape, lambda i: (0, 0))],
        out_specs=pl.BlockSpec((_BLK_A, _D), lambda i: (i, 0)),
        out_shape=jax.ShapeDtypeStruct((_N, _D), table.dtype),
    )(tab1)


def _pass_b_body(t_ref, tab_ref, buf_ref, out_ref):
    i = pl.program_id(0)
    t = t_ref[0]
    last = jnp.maximum(pl.cdiv(t, _BLK_B) - 1, 0)

    @pl.when(i < last)
    def _():
        out_ref[...] = jnp.broadcast_to(tab_ref[0:1, :], (_BLK_B, _D))

    @pl.when(i == last)
    def _():
        row = i * _BLK_B + jax.lax.broadcasted_iota(jnp.int32, (_BLK_B, _D), 0)
        t0 = jnp.broadcast_to(tab_ref[0:1, :], (_BLK_B, _D))
        t1 = jnp.broadcast_to(tab_ref[1:2, :], (_BLK_B, _D))
        out_ref[...] = jnp.where(row < t, t0, t1)


def _pass_b(t, table, buf):
    grid_spec = pltpu.PrefetchScalarGridSpec(
        num_scalar_prefetch=1,
        grid=(_N // _BLK_B,),
        in_specs=[
            pl.BlockSpec(table.shape, lambda i, t: (0, 0)),
            pl.BlockSpec(memory_space=pl.ANY),
        ],
        out_specs=pl.BlockSpec(
            (_BLK_B, _D),
            lambda i, t: (jnp.minimum(i, jnp.maximum(pl.cdiv(t[0], _BLK_B) - 1, 0)), 0),
        ),
    )
    return pl.pallas_call(
        _pass_b_body,
        grid_spec=grid_spec,
        out_shape=jax.ShapeDtypeStruct((_N, _D), table.dtype),
        input_output_aliases={2: 0},
    )(t, table, buf)


def kernel(x, table):
    t = _sc_scan(x)
    buf = _pass_a(table)
    out = _pass_b(t, table, buf)
    return out.reshape(x.shape + (table.shape[1],))


# speedup vs baseline: 1.0928x; 1.0525x over previous
import jax
import jax.numpy as jnp
from jax.experimental import pallas as pl
from jax.experimental.pallas import tpu as pltpu
from jax.experimental.pallas import tpu_sc as plsc

_SEP = 102
_N = 32768
_D = 2048
_BLK = 1024
_SUBCORES = 32
_PER_SUB = _N // _SUBCORES
_LANES = 16


def _sc_scan(x):
    xr = x.reshape(_SUBCORES, _PER_SUB)

    @pl.kernel(
        out_type=jax.ShapeDtypeStruct((_SUBCORES, _LANES), jnp.int32),
        mesh=plsc.VectorSubcoreMesh(core_axis_name="c", subcore_axis_name="s"),
        scratch_types=[
            pltpu.VMEM((_PER_SUB,), jnp.int32),
            pltpu.VMEM((_LANES,), jnp.int32),
        ],
    )
    def scan_kernel(x_hbm, o_hbm, xv, accv):
        k = jax.lax.axis_index("c") * 16 + jax.lax.axis_index("s")
        pltpu.sync_copy(x_hbm.at[k], xv)
        lane = jax.lax.iota(jnp.int32, _LANES)
        base = k * _PER_SUB

        def body(i, acc):
            v = xv[pl.ds(i * _LANES, _LANES)]
            pos = base + i * _LANES + lane
            return jnp.minimum(acc, jnp.where(v == _SEP, pos, _N))

        accv[...] = jax.lax.fori_loop(
            0, _PER_SUB // _LANES, body, jnp.full((_LANES,), _N, jnp.int32)
        )
        pltpu.sync_copy(accv, o_hbm.at[k])

    return scan_kernel(xr)


def _write_body(part_ref, tab_ref, out_ref, t_ref):
    i = pl.program_id(0)

    @pl.when(i == 0)
    def _():
        t_ref[0] = jnp.min(part_ref[...])

    t = t_ref[0]
    row = i * _BLK + jax.lax.broadcasted_iota(jnp.int32, (_BLK, _D), 0)
    t0 = jnp.broadcast_to(tab_ref[0:1, :], (_BLK, _D))
    t1 = jnp.broadcast_to(tab_ref[1:2, :], (_BLK, _D))
    out_ref[...] = jnp.where(row >= t, t1, t0)


def kernel(x, table):
    partials = _sc_scan(x)
    out = pl.pallas_call(
        _write_body,
        grid=(_N // _BLK,),
        in_specs=[
            pl.BlockSpec(partials.shape, lambda i: (0, 0)),
            pl.BlockSpec(table.shape, lambda i: (0, 0)),
        ],
        out_specs=pl.BlockSpec((_BLK, _D), lambda i: (i, 0)),
        out_shape=jax.ShapeDtypeStruct((_N, _D), table.dtype),
        scratch_shapes=[pltpu.SMEM((1,), jnp.int32)],
    )(partials, table)
    return out.reshape(x.shape + (table.shape[1],))


# speedup vs baseline: 1.1059x; 1.0121x over previous
import jax
import jax.numpy as jnp
from jax.experimental import pallas as pl
from jax.experimental.pallas import tpu as pltpu
from jax.experimental.pallas import tpu_sc as plsc

_SEP = 102
_N = 32768
_D = 2048
_BLK = 1024
_SUBCORES = 16
_PER_SUB = _N // _SUBCORES
_LANES = 16


def _sc_scan(x):
    xr = x.reshape(_SUBCORES, _PER_SUB)

    @pl.kernel(
        out_type=jax.ShapeDtypeStruct((_SUBCORES, _LANES), jnp.int32),
        mesh=plsc.VectorSubcoreMesh(
            core_axis_name="c", subcore_axis_name="s", num_cores=1
        ),
        scratch_types=[
            pltpu.VMEM((_PER_SUB,), jnp.int32),
            pltpu.VMEM((_LANES,), jnp.int32),
        ],
    )
    def scan_kernel(x_hbm, o_hbm, xv, accv):
        k = jax.lax.axis_index("s")
        pltpu.sync_copy(x_hbm.at[k], xv)
        lane = jax.lax.iota(jnp.int32, _LANES)
        base = k * _PER_SUB

        def body(i, acc):
            v = xv[pl.ds(i * _LANES, _LANES)]
            pos = base + i * _LANES + lane
            return jnp.minimum(acc, jnp.where(v == _SEP, pos, _N))

        accv[...] = jax.lax.fori_loop(
            0, _PER_SUB // _LANES, body, jnp.full((_LANES,), _N, jnp.int32)
        )
        pltpu.sync_copy(accv, o_hbm.at[k])

    return scan_kernel(xr)


def _write_body(part_ref, tab_ref, out_ref, t_ref):
    i = pl.program_id(0)

    @pl.when(i == 0)
    def _():
        t_ref[0] = jnp.min(part_ref[...])

    t = t_ref[0]
    row = i * _BLK + jax.lax.broadcasted_iota(jnp.int32, (_BLK, _D), 0)
    t0 = jnp.broadcast_to(tab_ref[0:1, :], (_BLK, _D))
    t1 = jnp.broadcast_to(tab_ref[1:2, :], (_BLK, _D))
    out_ref[...] = jnp.where(row >= t, t1, t0)


def kernel(x, table):
    partials = _sc_scan(x)
    out = pl.pallas_call(
        _write_body,
        grid=(_N // _BLK,),
        in_specs=[
            pl.BlockSpec(partials.shape, lambda i: (0, 0)),
            pl.BlockSpec(table.shape, lambda i: (0, 0)),
        ],
        out_specs=pl.BlockSpec((_BLK, _D), lambda i: (i, 0)),
        out_shape=jax.ShapeDtypeStruct((_N, _D), table.dtype),
        scratch_shapes=[pltpu.SMEM((1,), jnp.int32)],
    )(partials, table)
    return out.reshape(x.shape + (table.shape[1],))
